# baseline (speedup 1.0000x reference)
import jax
import jax.numpy as jnp
from jax.experimental import pallas as pl
from jax.experimental.pallas import tpu as pltpu
from jax.experimental.pallas import tpu_sc as plsc

N_NODES = 10000
N_EDGES = 320000
D_FEAT = 128
HIDDEN = 256
D_WORDS = HIDDEN // 2

NODE_BLOCK = 2000
N_NODE_BLOCKS = N_NODES // NODE_BLOCK

GATHER_WINDOW = 256
N_GATHER_BLOCKS = N_EDGES // GATHER_WINDOW

EDGE_BLOCK = 2560
N_EDGE_BLOCKS = N_EDGES // EDGE_BLOCK
OUT_ROWS = N_EDGES // 128


def _tc_tables(x, W1a, W1b):
    def body(x_ref, w1a_ref, w1b_ref, t_ref):
        pid = pl.program_id(0)
        xb = x_ref[...].astype(jnp.bfloat16)

        @pl.when(pid < N_NODE_BLOCKS)
        def _():
            t_ref[...] = jnp.dot(
                xb, w1a_ref[...], preferred_element_type=jnp.float32
            ).astype(jnp.bfloat16)

        @pl.when(pid >= N_NODE_BLOCKS)
        def _():
            t_ref[...] = jnp.dot(
                xb, w1b_ref[...], preferred_element_type=jnp.float32
            ).astype(jnp.bfloat16)

    return pl.pallas_call(
        body,
        grid=(2 * N_NODE_BLOCKS,),
        in_specs=[
            pl.BlockSpec((NODE_BLOCK, D_FEAT), lambda i: (i % N_NODE_BLOCKS, 0)),
            pl.BlockSpec((D_FEAT, HIDDEN), lambda i: (0, 0)),
            pl.BlockSpec((D_FEAT, HIDDEN), lambda i: (0, 0)),
        ],
        out_specs=pl.BlockSpec((NODE_BLOCK, HIDDEN), lambda i: (i, 0)),
        out_shape=jax.ShapeDtypeStruct((2 * N_NODES, HIDDEN), jnp.bfloat16),
    )(x, W1a, W1b)


def _sc_gather(tw, idx2d):
    mesh = plsc.VectorSubcoreMesh(core_axis_name="core", subcore_axis_name="subcore")

    @pl.kernel(
        out_type=jax.ShapeDtypeStruct((N_EDGES, 2 * D_WORDS), jnp.float32),
        mesh=mesh,
    )
    def gather_kernel(t_hbm, i_hbm, o_hbm):
        def body(i_vmem, o_vmem):
            pltpu.sync_copy(t_hbm.at[i_vmem.at[0]], o_vmem)

        pltpu.emit_pipeline(
            body,
            grid=(2 * N_GATHER_BLOCKS,),
            in_specs=[
                pl.BlockSpec((1, GATHER_WINDOW), index_map=lambda i: (0, i))
            ],
            out_specs=[
                pl.BlockSpec(
                    (GATHER_WINDOW, D_WORDS),
                    index_map=lambda i: (i % N_GATHER_BLOCKS, i // N_GATHER_BLOCKS),
                )
            ],
            core_axis_name=("core", "subcore"),
            dimension_semantics=(pltpu.PARALLEL,),
        )(i_hbm, o_hbm)

    return gather_kernel(tw, idx2d)


def _tc_consume(g, b1, W2, b2):
    def body(g_ref, b1_ref, w2_ref, b2_ref, o_ref):
        gv = g_ref[...]
        s = gv[:, :HIDDEN].astype(jnp.float32) + gv[:, HIDDEN:].astype(jnp.float32)
        h = jnp.maximum(s + b1_ref[...], 0.0)
        r = (
            jnp.dot(
                h.astype(jnp.bfloat16),
                w2_ref[...],
                preferred_element_type=jnp.float32,
            )
            + b2_ref[...]
        )
        o_ref[...] = r.reshape(1, EDGE_BLOCK // 128, 128)

    return pl.pallas_call(
        body,
        grid=(N_EDGE_BLOCKS,),
        in_specs=[
            pl.BlockSpec((EDGE_BLOCK, 2 * HIDDEN), lambda i: (i, 0)),
            pl.BlockSpec((1, HIDDEN), lambda i: (0, 0)),
            pl.BlockSpec((HIDDEN, 1), lambda i: (0, 0)),
            pl.BlockSpec((1, 1), lambda i: (0, 0)),
        ],
        out_specs=pl.BlockSpec(
            (1, EDGE_BLOCK // 128, 128), lambda i: (i, 0, 0)
        ),
        out_shape=jax.ShapeDtypeStruct(
            (N_EDGE_BLOCKS, EDGE_BLOCK // 128, 128), jnp.float32
        ),
    )(g, b1, W2, b2)


def kernel(x, start, end, W1, b1, W2, b2):
    W1bf = W1.astype(jnp.bfloat16)
    T = _tc_tables(x, W1bf[:D_FEAT], W1bf[D_FEAT:])
    tw = jax.lax.bitcast_convert_type(
        T.reshape(2 * N_NODES, D_WORDS, 2), jnp.float32
    )
    idx2d = jnp.concatenate([start, end + N_NODES]).reshape(1, 2 * N_EDGES)
    gw = _sc_gather(tw, idx2d)
    g = jax.lax.bitcast_convert_type(gw, jnp.bfloat16).reshape(
        N_EDGES, 2 * HIDDEN
    )
    out = _tc_consume(
        g, b1.reshape(1, HIDDEN), W2.astype(jnp.bfloat16), b2.reshape(1, 1)
    )
    return out.reshape(N_EDGES, 1)


# speedup vs baseline: 7.3206x; 7.3206x over previous
import jax
import jax.numpy as jnp
from jax.experimental import pallas as pl
from jax.experimental.pallas import tpu as pltpu
from jax.experimental.pallas import tpu_sc as plsc

N_NODES = 10000
N_EDGES = 320000
D_FEAT = 128
HIDDEN = 256
HALF = HIDDEN // 2

NODE_BLOCK = 2000
N_NODE_BLOCKS = N_NODES // NODE_BLOCK

GATHER_WINDOW = 256
N_GATHER_BLOCKS = N_EDGES // GATHER_WINDOW

EDGE_BLOCK = 2560
N_EDGE_BLOCKS = N_EDGES // EDGE_BLOCK


def _pack_bf16_pair(lo, hi):
    def rne(v):
        u = jax.lax.bitcast_convert_type(v, jnp.uint32)
        return (u + jnp.uint32(0x7FFF) + ((u >> 16) & jnp.uint32(1))) >> 16

    return rne(lo) | (rne(hi) << 16)


def _tc_tables(x, W1a, W1b, b1):
    def body(x_ref, w1a_ref, w1b_ref, b1_ref, t_ref):
        pid = pl.program_id(0)
        xb = x_ref[...].astype(jnp.bfloat16)

        @pl.when(pid < N_NODE_BLOCKS)
        def _():
            r = jnp.dot(xb, w1a_ref[...], preferred_element_type=jnp.float32)
            t_ref[...] = _pack_bf16_pair(r[:, :HALF], r[:, HALF:])

        @pl.when(pid >= N_NODE_BLOCKS)
        def _():
            r = (
                jnp.dot(xb, w1b_ref[...], preferred_element_type=jnp.float32)
                + b1_ref[...]
            )
            t_ref[...] = _pack_bf16_pair(r[:, :HALF], r[:, HALF:])

    return pl.pallas_call(
        body,
        grid=(2 * N_NODE_BLOCKS,),
        in_specs=[
            pl.BlockSpec((NODE_BLOCK, D_FEAT), lambda i: (i % N_NODE_BLOCKS, 0)),
            pl.BlockSpec((D_FEAT, HIDDEN), lambda i: (0, 0)),
            pl.BlockSpec((D_FEAT, HIDDEN), lambda i: (0, 0)),
            pl.BlockSpec((1, HIDDEN), lambda i: (0, 0)),
        ],
        out_specs=pl.BlockSpec((NODE_BLOCK, HALF), lambda i: (i, 0)),
        out_shape=jax.ShapeDtypeStruct((2 * N_NODES, HALF), jnp.uint32),
    )(x, W1a, W1b, b1)


def _sc_gather(t, idx2d):
    mesh = plsc.VectorSubcoreMesh(core_axis_name="core", subcore_axis_name="subcore")

    @pl.kernel(
        out_type=jax.ShapeDtypeStruct((N_EDGES, 2 * HALF), jnp.uint32),
        mesh=mesh,
    )
    def gather_kernel(t_hbm, i_hbm, o_hbm):
        def body(i_vmem, o_vmem):
            pltpu.sync_copy(t_hbm.at[i_vmem.at[0]], o_vmem)

        pltpu.emit_pipeline(
            body,
            grid=(2 * N_GATHER_BLOCKS,),
            in_specs=[
                pl.BlockSpec((1, GATHER_WINDOW), index_map=lambda i: (0, i))
            ],
            out_specs=[
                pl.BlockSpec(
                    (GATHER_WINDOW, HALF),
                    index_map=lambda i: (i % N_GATHER_BLOCKS, i // N_GATHER_BLOCKS),
                )
            ],
            core_axis_name=("core", "subcore"),
            dimension_semantics=(pltpu.PARALLEL,),
        )(i_hbm, o_hbm)

    return gather_kernel(t, idx2d)


def _tc_consume(g, W2, b2):
    def body(g_ref, w2_ref, b2_ref, o_ref):
        gv = g_ref[...]
        ua = gv[:, :HALF]
        ub = gv[:, HALF:]

        def unpack_lo(u):
            return jax.lax.bitcast_convert_type(u << 16, jnp.float32)

        def unpack_hi(u):
            return jax.lax.bitcast_convert_type(
                u & jnp.uint32(0xFFFF0000), jnp.float32
            )

        h_lo = jnp.maximum(unpack_lo(ua) + unpack_lo(ub), 0.0)
        h_hi = jnp.maximum(unpack_hi(ua) + unpack_hi(ub), 0.0)
        r = (
            jnp.dot(
                h_lo.astype(jnp.bfloat16),
                w2_ref[:HALF],
                preferred_element_type=jnp.float32,
            )
            + jnp.dot(
                h_hi.astype(jnp.bfloat16),
                w2_ref[HALF:],
                preferred_element_type=jnp.float32,
            )
            + b2_ref[...]
        )
        o_ref[...] = r.reshape(1, EDGE_BLOCK // 128, 128)

    return pl.pallas_call(
        body,
        grid=(N_EDGE_BLOCKS,),
        in_specs=[
            pl.BlockSpec((EDGE_BLOCK, 2 * HALF), lambda i: (i, 0)),
            pl.BlockSpec((HIDDEN, 1), lambda i: (0, 0)),
            pl.BlockSpec((1, 1), lambda i: (0, 0)),
        ],
        out_specs=pl.BlockSpec(
            (1, EDGE_BLOCK // 128, 128), lambda i: (i, 0, 0)
        ),
        out_shape=jax.ShapeDtypeStruct(
            (N_EDGE_BLOCKS, EDGE_BLOCK // 128, 128), jnp.float32
        ),
    )(g, W2, b2)


def kernel(x, start, end, W1, b1, W2, b2):
    W1bf = W1.astype(jnp.bfloat16)
    T = _tc_tables(x, W1bf[:D_FEAT], W1bf[D_FEAT:], b1.reshape(1, HIDDEN))
    idx2d = jnp.concatenate([start, end + N_NODES]).reshape(1, 2 * N_EDGES)
    g = _sc_gather(T, idx2d)
    out = _tc_consume(g, W2.astype(jnp.bfloat16), b2.reshape(1, 1))
    return out.reshape(N_EDGES, 1)


# speedup vs baseline: 8.1115x; 1.1080x over previous
import jax
import jax.numpy as jnp
from jax.experimental import pallas as pl
from jax.experimental.pallas import tpu as pltpu
from jax.experimental.pallas import tpu_sc as plsc

N_NODES = 10000
N_EDGES = 320000
D_FEAT = 128
HIDDEN = 256
HALF = HIDDEN // 2

NODE_BLOCK = 2000
N_NODE_BLOCKS = N_NODES // NODE_BLOCK

N_CHUNKS = 5
CHUNK = N_EDGES // N_CHUNKS

GATHER_WINDOW = 256
N_GATHER_BLOCKS = CHUNK // GATHER_WINDOW

EDGE_BLOCK = 6400
N_EDGE_BLOCKS = CHUNK // EDGE_BLOCK
OUT_SUB = EDGE_BLOCK // 128


def _pack_bf16_pair(lo, hi):
    def rne(v):
        u = jax.lax.bitcast_convert_type(v, jnp.uint32)
        return (u + jnp.uint32(0x7FFF) + ((u >> 16) & jnp.uint32(1))) >> 16

    return rne(lo) | (rne(hi) << 16)


def _tc_tables(x, W1a, W1b, b1):
    def body(x_ref, w1a_ref, w1b_ref, b1_ref, t_ref):
        pid = pl.program_id(0)
        xb = x_ref[...].astype(jnp.bfloat16)

        @pl.when(pid < N_NODE_BLOCKS)
        def _():
            r = jnp.dot(xb, w1a_ref[...], preferred_element_type=jnp.float32)
            t_ref[...] = _pack_bf16_pair(r[:, :HALF], r[:, HALF:])

        @pl.when(pid >= N_NODE_BLOCKS)
        def _():
            r = (
                jnp.dot(xb, w1b_ref[...], preferred_element_type=jnp.float32)
                + b1_ref[...]
            )
            t_ref[...] = _pack_bf16_pair(r[:, :HALF], r[:, HALF:])

    return pl.pallas_call(
        body,
        grid=(2 * N_NODE_BLOCKS,),
        in_specs=[
            pl.BlockSpec((NODE_BLOCK, D_FEAT), lambda i: (i % N_NODE_BLOCKS, 0)),
            pl.BlockSpec((D_FEAT, HIDDEN), lambda i: (0, 0)),
            pl.BlockSpec((D_FEAT, HIDDEN), lambda i: (0, 0)),
            pl.BlockSpec((1, HIDDEN), lambda i: (0, 0)),
        ],
        out_specs=pl.BlockSpec((NODE_BLOCK, HALF), lambda i: (i, 0)),
        out_shape=jax.ShapeDtypeStruct((2 * N_NODES, HALF), jnp.uint32),
    )(x, W1a, W1b, b1)


def _sc_gather(t, idx2d):
    mesh = plsc.VectorSubcoreMesh(core_axis_name="core", subcore_axis_name="subcore")

    @pl.kernel(
        out_type=jax.ShapeDtypeStruct((CHUNK, 2 * HALF), jnp.uint32),
        mesh=mesh,
    )
    def gather_kernel(t_hbm, i_hbm, o_hbm):
        def body(i_vmem, o_vmem):
            pltpu.sync_copy(t_hbm.at[i_vmem.at[0]], o_vmem)

        pltpu.emit_pipeline(
            body,
            grid=(2 * N_GATHER_BLOCKS,),
            in_specs=[
                pl.BlockSpec((1, GATHER_WINDOW), index_map=lambda i: (0, i))
            ],
            out_specs=[
                pl.BlockSpec(
                    (GATHER_WINDOW, HALF),
                    index_map=lambda i: (i % N_GATHER_BLOCKS, i // N_GATHER_BLOCKS),
                )
            ],
            core_axis_name=("core", "subcore"),
            dimension_semantics=(pltpu.PARALLEL,),
        )(i_hbm, o_hbm)

    return gather_kernel(t, idx2d)


def _tc_consume(g, W2, b2):
    def body(g_ref, w2_ref, b2_ref, o_ref):
        gv = g_ref[...]
        ua = gv[:, :HALF]
        ub = gv[:, HALF:]

        def unpack_lo(u):
            return jax.lax.bitcast_convert_type(u << 16, jnp.float32)

        def unpack_hi(u):
            return jax.lax.bitcast_convert_type(
                u & jnp.uint32(0xFFFF0000), jnp.float32
            )

        h_lo = jnp.maximum(unpack_lo(ua) + unpack_lo(ub), 0.0)
        h_hi = jnp.maximum(unpack_hi(ua) + unpack_hi(ub), 0.0)
        r = (
            jnp.dot(
                h_lo.astype(jnp.bfloat16),
                w2_ref[:HALF],
                preferred_element_type=jnp.float32,
            )
            + jnp.dot(
                h_hi.astype(jnp.bfloat16),
                w2_ref[HALF:],
                preferred_element_type=jnp.float32,
            )
            + b2_ref[...]
        )
        o_ref[...] = r.reshape(1, OUT_SUB, 128)

    return pl.pallas_call(
        body,
        grid=(N_EDGE_BLOCKS,),
        in_specs=[
            pl.BlockSpec((EDGE_BLOCK, 2 * HALF), lambda i: (i, 0)),
            pl.BlockSpec((HIDDEN, 1), lambda i: (0, 0)),
            pl.BlockSpec((1, 1), lambda i: (0, 0)),
        ],
        out_specs=pl.BlockSpec((1, OUT_SUB, 128), lambda i: (i, 0, 0)),
        out_shape=jax.ShapeDtypeStruct(
            (N_EDGE_BLOCKS, OUT_SUB, 128), jnp.float32
        ),
    )(g, W2, b2)


def kernel(x, start, end, W1, b1, W2, b2):
    W1bf = W1.astype(jnp.bfloat16)
    T = _tc_tables(x, W1bf[:D_FEAT], W1bf[D_FEAT:], b1.reshape(1, HIDDEN))
    W2bf = W2.astype(jnp.bfloat16)
    b2r = b2.reshape(1, 1)
    ends = end + N_NODES
    outs = []
    for c in range(N_CHUNKS):
        lo, hi = c * CHUNK, (c + 1) * CHUNK
        idx2d = jnp.concatenate([start[lo:hi], ends[lo:hi]]).reshape(1, 2 * CHUNK)
        g = _sc_gather(T, idx2d)
        outs.append(_tc_consume(g, W2bf, b2r))
    return jnp.concatenate(outs, axis=0).reshape(N_EDGES, 1)


# speedup vs baseline: 8.2319x; 1.0148x over previous
import jax
import jax.numpy as jnp
from jax.experimental import pallas as pl
from jax.experimental.pallas import tpu as pltpu
from jax.experimental.pallas import tpu_sc as plsc

N_NODES = 10000
N_EDGES = 320000
D_FEAT = 128
HIDDEN = 256
HALF = HIDDEN // 2

NODE_BLOCK = 2000
N_NODE_BLOCKS = N_NODES // NODE_BLOCK

N_CHUNKS = 5
CHUNK = N_EDGES // N_CHUNKS

GATHER_WINDOW = 256
N_GATHER_BLOCKS = CHUNK // GATHER_WINDOW

EDGE_BLOCK = 12800
N_EDGE_BLOCKS = CHUNK // EDGE_BLOCK
OUT_SUB = EDGE_BLOCK // 128


def _pack_bf16_pair(lo, hi):
    def rne(v):
        u = jax.lax.bitcast_convert_type(v, jnp.uint32)
        return (u + jnp.uint32(0x7FFF) + ((u >> 16) & jnp.uint32(1))) >> 16

    return rne(lo) | (rne(hi) << 16)


def _tc_tables(x, W1a, W1b, b1):
    def body(x_ref, w1a_ref, w1b_ref, b1_ref, t_ref):
        pid = pl.program_id(0)
        xb = x_ref[...].astype(jnp.bfloat16)

        @pl.when(pid < N_NODE_BLOCKS)
        def _():
            r = jnp.dot(xb, w1a_ref[...], preferred_element_type=jnp.float32)
            t_ref[...] = _pack_bf16_pair(r[:, :HALF], r[:, HALF:])

        @pl.when(pid >= N_NODE_BLOCKS)
        def _():
            r = (
                jnp.dot(xb, w1b_ref[...], preferred_element_type=jnp.float32)
                + b1_ref[...]
            )
            t_ref[...] = _pack_bf16_pair(r[:, :HALF], r[:, HALF:])

    return pl.pallas_call(
        body,
        grid=(2 * N_NODE_BLOCKS,),
        in_specs=[
            pl.BlockSpec((NODE_BLOCK, D_FEAT), lambda i: (i % N_NODE_BLOCKS, 0)),
            pl.BlockSpec((D_FEAT, HIDDEN), lambda i: (0, 0)),
            pl.BlockSpec((D_FEAT, HIDDEN), lambda i: (0, 0)),
            pl.BlockSpec((1, HIDDEN), lambda i: (0, 0)),
        ],
        out_specs=pl.BlockSpec((NODE_BLOCK, HALF), lambda i: (i, 0)),
        out_shape=jax.ShapeDtypeStruct((2 * N_NODES, HALF), jnp.uint32),
    )(x, W1a, W1b, b1)


def _sc_gather(t, idx2d):
    mesh = plsc.VectorSubcoreMesh(core_axis_name="core", subcore_axis_name="subcore")

    @pl.kernel(
        out_type=jax.ShapeDtypeStruct((CHUNK, 2 * HALF), jnp.uint32),
        mesh=mesh,
    )
    def gather_kernel(t_hbm, i_hbm, o_hbm):
        def body(i_vmem, o_vmem):
            pltpu.sync_copy(t_hbm.at[i_vmem.at[0]], o_vmem)

        pltpu.emit_pipeline(
            body,
            grid=(2 * N_GATHER_BLOCKS,),
            in_specs=[
                pl.BlockSpec((1, GATHER_WINDOW), index_map=lambda i: (0, i))
            ],
            out_specs=[
                pl.BlockSpec(
                    (GATHER_WINDOW, HALF),
                    index_map=lambda i: (i % N_GATHER_BLOCKS, i // N_GATHER_BLOCKS),
                )
            ],
            core_axis_name=("core", "subcore"),
            dimension_semantics=(pltpu.PARALLEL,),
        )(i_hbm, o_hbm)

    return gather_kernel(t, idx2d)


def _tc_consume(g, W2, b2):
    def body(g_ref, w2_ref, b2_ref, o_ref):
        gv = g_ref[...]
        ua = gv[:, :HALF]
        ub = gv[:, HALF:]

        def unpack_lo(u):
            return jax.lax.bitcast_convert_type(u << 16, jnp.float32)

        def unpack_hi(u):
            return jax.lax.bitcast_convert_type(
                u & jnp.uint32(0xFFFF0000), jnp.float32
            )

        h_lo = jnp.maximum(unpack_lo(ua) + unpack_lo(ub), 0.0)
        h_hi = jnp.maximum(unpack_hi(ua) + unpack_hi(ub), 0.0)
        r = (
            jnp.dot(
                h_lo.astype(jnp.bfloat16),
                w2_ref[:HALF],
                preferred_element_type=jnp.float32,
            )
            + jnp.dot(
                h_hi.astype(jnp.bfloat16),
                w2_ref[HALF:],
                preferred_element_type=jnp.float32,
            )
            + b2_ref[...]
        )
        o_ref[...] = r.reshape(1, OUT_SUB, 128)

    return pl.pallas_call(
        body,
        grid=(N_EDGE_BLOCKS,),
        in_specs=[
            pl.BlockSpec((EDGE_BLOCK, 2 * HALF), lambda i: (i, 0)),
            pl.BlockSpec((HIDDEN, 1), lambda i: (0, 0)),
            pl.BlockSpec((1, 1), lambda i: (0, 0)),
        ],
        out_specs=pl.BlockSpec((1, OUT_SUB, 128), lambda i: (i, 0, 0)),
        out_shape=jax.ShapeDtypeStruct(
            (N_EDGE_BLOCKS, OUT_SUB, 128), jnp.float32
        ),
    )(g, W2, b2)


def kernel(x, start, end, W1, b1, W2, b2):
    W1bf = W1.astype(jnp.bfloat16)
    T = _tc_tables(x, W1bf[:D_FEAT], W1bf[D_FEAT:], b1.reshape(1, HIDDEN))
    W2bf = W2.astype(jnp.bfloat16)
    b2r = b2.reshape(1, 1)
    ends = end + N_NODES
    outs = []
    for c in range(N_CHUNKS):
        lo, hi = c * CHUNK, (c + 1) * CHUNK
        idx2d = jnp.concatenate([start[lo:hi], ends[lo:hi]]).reshape(1, 2 * CHUNK)
        g = _sc_gather(T, idx2d)
        outs.append(_tc_consume(g, W2bf, b2r))
    return jnp.concatenate(outs, axis=0).reshape(N_EDGES, 1)


# speedup vs baseline: 8.3200x; 1.0107x over previous
import jax
import jax.numpy as jnp
from jax.experimental import pallas as pl
from jax.experimental.pallas import tpu as pltpu
from jax.experimental.pallas import tpu_sc as plsc

N_NODES = 10000
N_EDGES = 320000
D_FEAT = 128
HIDDEN = 256
HALF = HIDDEN // 2

NODE_BLOCK = 5000
N_NODE_BLOCKS = N_NODES // NODE_BLOCK

CHUNK_SIZES = (76800, 76800, 76800, 76800, 12800)

GATHER_WINDOW = 256

EDGE_BLOCK = 12800
OUT_SUB = EDGE_BLOCK // 128


def _pack_bf16_pair(lo, hi):
    def rne(v):
        u = jax.lax.bitcast_convert_type(v, jnp.uint32)
        return (u + jnp.uint32(0x7FFF) + ((u >> 16) & jnp.uint32(1))) >> 16

    return rne(lo) | (rne(hi) << 16)


def _tc_tables(x, W1a, W1b, b1):
    def body(x_ref, w1a_ref, w1b_ref, b1_ref, t_ref):
        pid = pl.program_id(0)
        xb = x_ref[...].astype(jnp.bfloat16)

        @pl.when(pid < N_NODE_BLOCKS)
        def _():
            r = jnp.dot(xb, w1a_ref[...], preferred_element_type=jnp.float32)
            t_ref[...] = _pack_bf16_pair(r[:, :HALF], r[:, HALF:])

        @pl.when(pid >= N_NODE_BLOCKS)
        def _():
            r = (
                jnp.dot(xb, w1b_ref[...], preferred_element_type=jnp.float32)
                + b1_ref[...]
            )
            t_ref[...] = _pack_bf16_pair(r[:, :HALF], r[:, HALF:])

    return pl.pallas_call(
        body,
        grid=(2 * N_NODE_BLOCKS,),
        in_specs=[
            pl.BlockSpec((NODE_BLOCK, D_FEAT), lambda i: (i % N_NODE_BLOCKS, 0)),
            pl.BlockSpec((D_FEAT, HIDDEN), lambda i: (0, 0)),
            pl.BlockSpec((D_FEAT, HIDDEN), lambda i: (0, 0)),
            pl.BlockSpec((1, HIDDEN), lambda i: (0, 0)),
        ],
        out_specs=pl.BlockSpec((NODE_BLOCK, HALF), lambda i: (i, 0)),
        out_shape=jax.ShapeDtypeStruct((2 * N_NODES, HALF), jnp.uint32),
    )(x, W1a, W1b, b1)


def _sc_gather(t, idx2d, chunk):
    mesh = plsc.VectorSubcoreMesh(core_axis_name="core", subcore_axis_name="subcore")
    nb = chunk // GATHER_WINDOW

    @pl.kernel(
        out_type=jax.ShapeDtypeStruct((chunk, 2 * HALF), jnp.uint32),
        mesh=mesh,
    )
    def gather_kernel(t_hbm, i_hbm, o_hbm):
        def body(i_vmem, o_vmem):
            pltpu.sync_copy(t_hbm.at[i_vmem.at[0]], o_vmem)

        pltpu.emit_pipeline(
            body,
            grid=(2 * nb,),
            in_specs=[
                pl.BlockSpec((1, GATHER_WINDOW), index_map=lambda i: (0, i))
            ],
            out_specs=[
                pl.BlockSpec(
                    (GATHER_WINDOW, HALF),
                    index_map=lambda i, nb=nb: (i % nb, i // nb),
                )
            ],
            core_axis_name=("core", "subcore"),
            dimension_semantics=(pltpu.PARALLEL,),
        )(i_hbm, o_hbm)

    return gather_kernel(t, idx2d)


def _tc_consume(g, W2, b2):
    def body(g_ref, w2_ref, b2_ref, o_ref):
        gv = g_ref[...]
        ua = gv[:, :HALF]
        ub = gv[:, HALF:]

        def unpack_lo(u):
            return jax.lax.bitcast_convert_type(u << 16, jnp.float32)

        def unpack_hi(u):
            return jax.lax.bitcast_convert_type(
                u & jnp.uint32(0xFFFF0000), jnp.float32
            )

        h_lo = jnp.maximum(unpack_lo(ua) + unpack_lo(ub), 0.0)
        h_hi = jnp.maximum(unpack_hi(ua) + unpack_hi(ub), 0.0)
        r = (
            jnp.dot(
                h_lo.astype(jnp.bfloat16),
                w2_ref[:HALF],
                preferred_element_type=jnp.float32,
            )
            + jnp.dot(
                h_hi.astype(jnp.bfloat16),
                w2_ref[HALF:],
                preferred_element_type=jnp.float32,
            )
            + b2_ref[...]
        )
        o_ref[...] = r.reshape(1, OUT_SUB, 128)

    n_blocks = g.shape[0] // EDGE_BLOCK
    return pl.pallas_call(
        body,
        grid=(n_blocks,),
        in_specs=[
            pl.BlockSpec((EDGE_BLOCK, 2 * HALF), lambda i: (i, 0)),
            pl.BlockSpec((HIDDEN, 1), lambda i: (0, 0)),
            pl.BlockSpec((1, 1), lambda i: (0, 0)),
        ],
        out_specs=pl.BlockSpec((1, OUT_SUB, 128), lambda i: (i, 0, 0)),
        out_shape=jax.ShapeDtypeStruct((n_blocks, OUT_SUB, 128), jnp.float32),
    )(g, W2, b2)


def kernel(x, start, end, W1, b1, W2, b2):
    W1bf = W1.astype(jnp.bfloat16)
    T = _tc_tables(x, W1bf[:D_FEAT], W1bf[D_FEAT:], b1.reshape(1, HIDDEN))
    W2bf = W2.astype(jnp.bfloat16)
    b2r = b2.reshape(1, 1)
    ends = end + N_NODES
    outs = []
    lo = 0
    for chunk in CHUNK_SIZES:
        hi = lo + chunk
        idx2d = jnp.concatenate([start[lo:hi], ends[lo:hi]]).reshape(1, 2 * chunk)
        g = _sc_gather(T, idx2d, chunk)
        outs.append(_tc_consume(g, W2bf, b2r))
        lo = hi
    return jnp.concatenate(outs, axis=0).reshape(N_EDGES, 1)


# speedup vs baseline: 8.4136x; 1.0113x over previous
import jax
import jax.numpy as jnp
from jax.experimental import pallas as pl
from jax.experimental.pallas import tpu as pltpu
from jax.experimental.pallas import tpu_sc as plsc

N_NODES = 10000
N_EDGES = 320000
D_FEAT = 128
HIDDEN = 256
HALF = HIDDEN // 2

NODE_BLOCK = 5000
N_NODE_BLOCKS = N_NODES // NODE_BLOCK

CHUNK_SIZES = (102400, 102400, 102400, 12800)

GATHER_WINDOW = 256

EDGE_BLOCK = 12800
OUT_SUB = EDGE_BLOCK // 128


def _pack_bf16_pair(lo, hi):
    def rne(v):
        u = jax.lax.bitcast_convert_type(v, jnp.uint32)
        return (u + jnp.uint32(0x7FFF) + ((u >> 16) & jnp.uint32(1))) >> 16

    return rne(lo) | (rne(hi) << 16)


def _tc_tables(x, W1a, W1b, b1):
    def body(x_ref, w1a_ref, w1b_ref, b1_ref, t_ref):
        pid = pl.program_id(0)
        xb = x_ref[...].astype(jnp.bfloat16)

        @pl.when(pid < N_NODE_BLOCKS)
        def _():
            r = jnp.dot(xb, w1a_ref[...], preferred_element_type=jnp.float32)
            t_ref[...] = _pack_bf16_pair(r[:, :HALF], r[:, HALF:])

        @pl.when(pid >= N_NODE_BLOCKS)
        def _():
            r = (
                jnp.dot(xb, w1b_ref[...], preferred_element_type=jnp.float32)
                + b1_ref[...]
            )
            t_ref[...] = _pack_bf16_pair(r[:, :HALF], r[:, HALF:])

    return pl.pallas_call(
        body,
        grid=(2 * N_NODE_BLOCKS,),
        in_specs=[
            pl.BlockSpec((NODE_BLOCK, D_FEAT), lambda i: (i % N_NODE_BLOCKS, 0)),
            pl.BlockSpec((D_FEAT, HIDDEN), lambda i: (0, 0)),
            pl.BlockSpec((D_FEAT, HIDDEN), lambda i: (0, 0)),
            pl.BlockSpec((1, HIDDEN), lambda i: (0, 0)),
        ],
        out_specs=pl.BlockSpec((NODE_BLOCK, HALF), lambda i: (i, 0)),
        out_shape=jax.ShapeDtypeStruct((2 * N_NODES, HALF), jnp.uint32),
    )(x, W1a, W1b, b1)


def _sc_gather(t, idx2d, chunk):
    mesh = plsc.VectorSubcoreMesh(core_axis_name="core", subcore_axis_name="subcore")
    nb = chunk // GATHER_WINDOW

    @pl.kernel(
        out_type=jax.ShapeDtypeStruct((chunk, 2 * HALF), jnp.uint32),
        mesh=mesh,
    )
    def gather_kernel(t_hbm, i_hbm, o_hbm):
        def body(i_vmem, o_vmem):
            pltpu.sync_copy(t_hbm.at[i_vmem.at[0]], o_vmem)

        pltpu.emit_pipeline(
            body,
            grid=(2 * nb,),
            in_specs=[
                pl.BlockSpec((1, GATHER_WINDOW), index_map=lambda i: (0, i))
            ],
            out_specs=[
                pl.BlockSpec(
                    (GATHER_WINDOW, HALF),
                    index_map=lambda i, nb=nb: (i % nb, i // nb),
                )
            ],
            core_axis_name=("core", "subcore"),
            dimension_semantics=(pltpu.PARALLEL,),
        )(i_hbm, o_hbm)

    return gather_kernel(t, idx2d)


def _tc_consume(g, W2, b2):
    def body(g_ref, w2_ref, b2_ref, o_ref):
        gv = g_ref[...]
        ua = gv[:, :HALF]
        ub = gv[:, HALF:]

        def unpack_lo(u):
            return jax.lax.bitcast_convert_type(u << 16, jnp.float32)

        def unpack_hi(u):
            return jax.lax.bitcast_convert_type(
                u & jnp.uint32(0xFFFF0000), jnp.float32
            )

        h_lo = jnp.maximum(unpack_lo(ua) + unpack_lo(ub), 0.0)
        h_hi = jnp.maximum(unpack_hi(ua) + unpack_hi(ub), 0.0)
        r = (
            jnp.dot(
                h_lo.astype(jnp.bfloat16),
                w2_ref[:HALF],
                preferred_element_type=jnp.float32,
            )
            + jnp.dot(
                h_hi.astype(jnp.bfloat16),
                w2_ref[HALF:],
                preferred_element_type=jnp.float32,
            )
            + b2_ref[...]
        )
        o_ref[...] = r.reshape(1, OUT_SUB, 128)

    n_blocks = g.shape[0] // EDGE_BLOCK
    return pl.pallas_call(
        body,
        grid=(n_blocks,),
        in_specs=[
            pl.BlockSpec((EDGE_BLOCK, 2 * HALF), lambda i: (i, 0)),
            pl.BlockSpec((HIDDEN, 1), lambda i: (0, 0)),
            pl.BlockSpec((1, 1), lambda i: (0, 0)),
        ],
        out_specs=pl.BlockSpec((1, OUT_SUB, 128), lambda i: (i, 0, 0)),
        out_shape=jax.ShapeDtypeStruct((n_blocks, OUT_SUB, 128), jnp.float32),
    )(g, W2, b2)


def kernel(x, start, end, W1, b1, W2, b2):
    W1bf = W1.astype(jnp.bfloat16)
    T = _tc_tables(x, W1bf[:D_FEAT], W1bf[D_FEAT:], b1.reshape(1, HIDDEN))
    W2bf = W2.astype(jnp.bfloat16)
    b2r = b2.reshape(1, 1)
    ends = end + N_NODES
    outs = []
    lo = 0
    for chunk in CHUNK_SIZES:
        hi = lo + chunk
        idx2d = jnp.concatenate([start[lo:hi], ends[lo:hi]]).reshape(1, 2 * chunk)
        g = _sc_gather(T, idx2d, chunk)
        outs.append(_tc_consume(g, W2bf, b2r))
        lo = hi
    return jnp.concatenate(outs, axis=0).reshape(N_EDGES, 1)
